# baseline (speedup 1.0000x reference)
import functools

import jax
import jax.numpy as jnp
from jax import lax
from jax.experimental import pallas as pl
from jax.experimental.pallas import tpu as pltpu
from jax.experimental.pallas import tpu_sc as plsc

NEG_SLOPE = 0.01
CH = 128
NC = 2
NS = 16
NW = NC * NS


_mesh = functools.partial(
    plsc.VectorSubcoreMesh, core_axis_name="c", subcore_axis_name="s")


def _make_k_deg(np_, h, slab, tail):
    q = 128 // h
    p = np_ // q
    stile = (-(-p // NS) + 7) // 8 * 8
    srem = p - stile * (NS - 1)
    assert 0 < srem <= stile
    nbuf = -(-stile // 16) * 16

    @functools.partial(
        pl.kernel,
        out_type=jax.ShapeDtypeStruct((NC, p, 128), jnp.float32),
        mesh=_mesh(),
        scratch_types=[
            pltpu.VMEM((slab, CH), jnp.int32),
            pltpu.VMEM((slab, CH), jnp.float32),
            pltpu.VMEM((nbuf,), jnp.float32),
            pltpu.VMEM((nbuf, 128), jnp.float32),
            pltpu.VMEM_SHARED((np_,), jnp.float32),
            pltpu.SemaphoreType.DMA,
        ],
        compiler_params=pltpu.CompilerParams(use_tc_tiling_on_sc=False),
    )
    def k_deg(col_hbm, ew_hbm, zn_hbm, z2_hbm, deg_out, col_v, ew_v, dloc,
              rep, deg_sh, dsem):
        cid = lax.axis_index("c")
        sid = lax.axis_index("s")
        wid = sid * NC + cid

        @pl.when(sid == 0)
        def _():
            pltpu.sync_copy(zn_hbm, deg_sh)

        @pl.when(wid < NW - 1)
        def _():
            pltpu.sync_copy(col_hbm.at[pl.ds(wid * slab, slab)], col_v)
            pltpu.sync_copy(ew_hbm.at[pl.ds(wid * slab, slab)], ew_v)

        @pl.when(wid == NW - 1)
        def _():
            sl = pl.ds((NW - 1) * slab, tail)
            dst = pl.ds(0, tail)
            pltpu.sync_copy(col_hbm.at[sl], col_v.at[dst])
            pltpu.sync_copy(ew_hbm.at[sl], ew_v.at[dst])

        plsc.subcore_barrier()

        def body(ci, carry):
            pltpu.async_copy(ew_v.at[ci], deg_sh.at[col_v.at[ci]], dsem,
                             add=True)
            return carry

        def drain(ci, carry):
            pltpu.make_async_copy(ew_v.at[ci], deg_sh.at[col_v.at[ci]],
                                  dsem).wait()
            return carry

        @pl.when(wid < NW - 1)
        def _():
            lax.fori_loop(0, slab, body, 0)
            lax.fori_loop(0, slab, drain, 0)

        @pl.when(wid == NW - 1)
        def _():
            lax.fori_loop(0, tail, body, 0)
            lax.fori_loop(0, tail, drain, 0)

        plsc.subcore_barrier()

        def replicate(off, cnt):
            pltpu.sync_copy(z2_hbm, rep)
            for i in range(q):
                pltpu.sync_copy(deg_sh.at[pl.ds(i * p + off, cnt)],
                                dloc.at[pl.ds(0, cnt)])

                def rep16(g, carry):
                    d16 = dloc[pl.ds(g * 16, 16)]
                    for j in range(16):
                        s = d16[j]
                        r = g * 16 + j
                        for k in range(h // 16):
                            sl = pl.ds(i * h + k * 16, 16)
                            rep[r, sl] = rep[r, sl] + s
                    return carry

                lax.fori_loop(0, -(-cnt // 16), rep16, 0)
            pltpu.sync_copy(rep.at[pl.ds(0, cnt)],
                            deg_out.at[cid, pl.ds(off, cnt)])

        @pl.when(sid < NS - 1)
        def _():
            replicate(sid * stile, stile)

        @pl.when(sid == NS - 1)
        def _():
            replicate((NS - 1) * stile, srem)

    return k_deg


def _make_k_prop(np_, h, slab, tail):
    q = 128 // h
    p = np_ // q
    stile = (-(-p // NS) + 7) // 8 * 8
    srem = p - stile * (NS - 1)
    assert 0 < srem <= stile

    @functools.partial(
        pl.kernel,
        out_type=jax.ShapeDtypeStruct((NC, p, 128), jnp.float32),
        mesh=_mesh(),
        scratch_types=[
            pltpu.VMEM((slab, CH), jnp.int32),
            pltpu.VMEM((slab, CH), jnp.int32),
            pltpu.VMEM((slab, CH), jnp.float32),
            [pltpu.VMEM((CH, h), jnp.float32)] * 8,
            pltpu.VMEM_SHARED((np_, h), jnp.float32),
            pltpu.VMEM_SHARED((np_, h), jnp.float32),
            [pltpu.SemaphoreType.DMA] * 8,
            [pltpu.SemaphoreType.DMA] * 8,
        ],
        compiler_params=pltpu.CompilerParams(use_tc_tiling_on_sc=False),
    )
    def k_prop(g_hbm, row_hbm, col_hbm, ew_hbm, zacc_hbm, out_hbm,
               row_v, col_v, ew_v, rows, acc_sh, g_sh, gsem, ssem):
        cid = lax.axis_index("c")
        sid = lax.axis_index("s")
        wid = sid * NC + cid

        def stage(off, cnt):
            pltpu.sync_copy(zacc_hbm.at[pl.ds(off * q, cnt * q)],
                            acc_sh.at[pl.ds(off * q, cnt * q)])
            for i in range(q):
                pltpu.sync_copy(g_hbm.at[pl.ds(off, cnt), pl.ds(i * h, h)],
                                g_sh.at[pl.ds(i * p + off, cnt)])

        @pl.when(sid < NS - 1)
        def _():
            stage(sid * stile, stile)

        @pl.when(sid == NS - 1)
        def _():
            stage((NS - 1) * stile, srem)

        @pl.when(wid < NW - 1)
        def _():
            sl = pl.ds(wid * slab, slab)
            pltpu.sync_copy(row_hbm.at[sl], row_v)
            pltpu.sync_copy(col_hbm.at[sl], col_v)
            pltpu.sync_copy(ew_hbm.at[sl], ew_v)

        @pl.when(wid == NW - 1)
        def _():
            sl = pl.ds((NW - 1) * slab, tail)
            dst = pl.ds(0, tail)
            pltpu.sync_copy(row_hbm.at[sl], row_v.at[dst])
            pltpu.sync_copy(col_hbm.at[sl], col_v.at[dst])
            pltpu.sync_copy(ew_hbm.at[sl], ew_v.at[dst])

        plsc.subcore_barrier()

        def scale(ci, rows_v):
            def scale16(q, c2):
                ew16 = ew_v[ci, pl.ds(q * 16, 16)]
                for j in range(16):
                    s = ew16[j]
                    r = q * 16 + j
                    for k in range(h // 16):
                        sl = pl.ds(k * 16, 16)
                        rows_v[r, sl] = rows_v[r, sl] * s
                return c2

            lax.fori_loop(0, CH // 16, scale16, 0)

        def gather(ci, b):
            pltpu.async_copy(g_sh.at[row_v.at[ci]], rows[b], gsem[b])

        def wait_gather(ci, b):
            pltpu.make_async_copy(g_sh.at[row_v.at[ci]], rows[b], gsem[b]).wait()

        def scatter(ci, b):
            pltpu.async_copy(rows[b], acc_sh.at[col_v.at[ci]], ssem[b],
                             add=True)

        def wait_scatter(ci, b):
            pltpu.make_async_copy(rows[b], acc_sh.at[col_v.at[ci]],
                                  ssem[b]).wait()

        def pipeline(nck):
            for j in range(4):
                gather(j, j)

            def body(i, carry):
                for k in range(8):
                    c = 8 * i + k
                    b = k
                    bn = (k + 4) % 8
                    wait_gather(c, b)
                    scale(c, rows[b])
                    scatter(c, b)

                    @pl.when(c + 4 < nck)
                    def _():
                        @pl.when(c >= 4)
                        def _():
                            wait_scatter(c - 4, bn)

                        gather(c + 4, bn)

                return carry

            lax.fori_loop(0, nck // 8, body, 0)
            for c in range(nck - 8, nck):
                wait_scatter(c, c % 8)

        @pl.when(wid < NW - 1)
        def _():
            pipeline(slab)

        @pl.when(wid == NW - 1)
        def _():
            pipeline(tail)

        plsc.subcore_barrier()

        def copy_out(off, cnt):
            for i in range(q):
                pltpu.sync_copy(acc_sh.at[pl.ds(i * p + off, cnt)],
                                out_hbm.at[cid, pl.ds(off, cnt),
                                           pl.ds(i * h, h)])

        @pl.when(sid < NS - 1)
        def _():
            copy_out(sid * stile, stile)

        @pl.when(sid == NS - 1)
        def _():
            copy_out((NS - 1) * stile, srem)

    return k_prop


def _tc1_body(n, np_, h, deg4_ref, x_ref, w1_ref, dis4_ref, g1_ref):
    q = 128 // h
    p = np_ // q
    dis4 = lax.rsqrt(deg4_ref[0] + deg4_ref[1] + 1.0)
    dis4_ref[...] = dis4
    x = x_ref[...]
    w1 = w1_ref[...]
    parts = []
    for i in range(q):
        lo = min(i * p, n)
        hi = min(i * p + p, n)
        hb = jnp.dot(x[lo:hi], w1, preferred_element_type=jnp.float32)
        parts.append(jnp.pad(hb, ((0, p - (hi - lo)), (0, 0))))
    h1p = jnp.concatenate(parts, axis=1)
    g1_ref[...] = dis4 * h1p


def _tc2_body(acc_ref, g1_ref, dis4_ref, b1_ref, w2_ref, g2_ref):
    dis4 = dis4_ref[...]
    pre = dis4 * (acc_ref[0] + acc_ref[1] + g1_ref[...]) + b1_ref[...]
    z = jnp.where(pre > 0, pre, NEG_SLOPE * pre)
    h2 = jnp.dot(z, w2_ref[...], preferred_element_type=jnp.float32)
    g2_ref[...] = h2 * dis4


def _tc3_body(acc_ref, g2_ref, dis4_ref, b2_ref, out_ref):
    out_ref[...] = (dis4_ref[...] * (acc_ref[0] + acc_ref[1] + g2_ref[...])
                    + b2_ref[...])


def kernel(x, edge_index, edge_weight, W1, b1, W2, b2):
    n, d = x.shape
    h = W1.shape[1]
    assert W2.shape[1] == h and 128 % h == 0
    e = edge_index.shape[1]
    q = 128 // h

    np_ = -(-n // (8 * q)) * (8 * q)
    p = np_ * h // 128
    assert n % q == 0

    chunks = -(-e // CH)
    slab = max(8, -(-(-(-chunks // NW)) // 8) * 8)
    tail = chunks - (NW - 1) * slab
    row_f = edge_index[0]
    col_f = edge_index[1]
    ew_f = edge_weight
    if e != chunks * CH or tail <= 0 or tail % 8 != 0:
        slab = max(8, -(-chunks // NW))
        slab = -(-slab // 8) * 8
        e_pad = NW * slab * CH
        tail = slab
        chunks = NW * slab
        row_f = jnp.pad(row_f, (0, e_pad - e))
        col_f = jnp.pad(col_f, (0, e_pad - e))
        ew_f = jnp.pad(ew_f, (0, e_pad - e))
    row_f = row_f.reshape(chunks, CH)
    col_f = col_f.reshape(chunks, CH)
    ew_f = ew_f.reshape(chunks, CH)

    zn = jnp.zeros((np_,), jnp.float32)
    zacc = jnp.zeros((np_, h), jnp.float32)
    stile = (-(-p // NS) + 7) // 8 * 8
    nbuf = -(-stile // 16) * 16
    z2 = jnp.zeros((nbuf, 128), jnp.float32)
    b1t = jnp.tile(b1, q).reshape(1, 128)
    b2t = jnp.tile(b2, q).reshape(1, 128)
    w2k = jnp.kron(jnp.eye(q, dtype=jnp.float32), W2)

    k_deg = _make_k_deg(np_, h, slab, tail)
    k_prop = _make_k_prop(np_, h, slab, tail)

    deg4 = k_deg(col_f, ew_f, zn, z2)

    k_tc1 = pl.pallas_call(
        functools.partial(_tc1_body, n, np_, h),
        out_shape=(jax.ShapeDtypeStruct((p, 128), jnp.float32),
                   jax.ShapeDtypeStruct((p, 128), jnp.float32)),
    )
    dis4, g1 = k_tc1(deg4, x, W1)

    acc1 = k_prop(g1, row_f, col_f, ew_f, zacc)

    k_tc2 = pl.pallas_call(
        _tc2_body,
        out_shape=jax.ShapeDtypeStruct((p, 128), jnp.float32),
    )
    g2 = k_tc2(acc1, g1, dis4, b1t, w2k)

    acc2 = k_prop(g2, row_f, col_f, ew_f, zacc)

    k_tc3 = pl.pallas_call(
        _tc3_body,
        out_shape=jax.ShapeDtypeStruct((p, 128), jnp.float32),
    )
    outp = k_tc3(acc2, g2, dis4, b2t)
    out = jnp.concatenate([outp[:, i * h:(i + 1) * h] for i in range(q)],
                          axis=0)
    return out[:n]


# speedup vs baseline: 1.0854x; 1.0854x over previous
import functools

import jax
import jax.numpy as jnp
from jax import lax
from jax.experimental import pallas as pl
from jax.experimental.pallas import tpu as pltpu
from jax.experimental.pallas import tpu_sc as plsc

NEG_SLOPE = 0.01
CH = 128
NC = 2
NS = 16
NW = NC * NS


_mesh = functools.partial(
    plsc.VectorSubcoreMesh, core_axis_name="c", subcore_axis_name="s")


def _make_k_deg(np_, h, slab, tail):
    q = 128 // h
    p = np_ // q
    stile = (-(-p // NS) + 7) // 8 * 8
    srem = p - stile * (NS - 1)
    assert 0 < srem <= stile
    nbuf = -(-stile // 16) * 16

    @functools.partial(
        pl.kernel,
        out_type=jax.ShapeDtypeStruct((NC, p, 128), jnp.float32),
        mesh=_mesh(),
        scratch_types=[
            pltpu.VMEM((slab, CH), jnp.int32),
            pltpu.VMEM((slab, CH), jnp.float32),
            pltpu.VMEM((nbuf,), jnp.float32),
            pltpu.VMEM((nbuf, 128), jnp.float32),
            pltpu.VMEM_SHARED((np_,), jnp.float32),
            pltpu.SemaphoreType.DMA,
        ],
        compiler_params=pltpu.CompilerParams(use_tc_tiling_on_sc=False),
    )
    def k_deg(col_hbm, ew_hbm, zn_hbm, z2_hbm, deg_out, col_v, ew_v, dloc,
              rep, deg_sh, dsem):
        cid = lax.axis_index("c")
        sid = lax.axis_index("s")
        wid = sid * NC + cid

        @pl.when(sid == 0)
        def _():
            pltpu.sync_copy(zn_hbm, deg_sh)

        @pl.when(wid < NW - 1)
        def _():
            pltpu.sync_copy(col_hbm.at[pl.ds(wid * slab, slab)], col_v)
            pltpu.sync_copy(ew_hbm.at[pl.ds(wid * slab, slab)], ew_v)

        @pl.when(wid == NW - 1)
        def _():
            sl = pl.ds((NW - 1) * slab, tail)
            dst = pl.ds(0, tail)
            pltpu.sync_copy(col_hbm.at[sl], col_v.at[dst])
            pltpu.sync_copy(ew_hbm.at[sl], ew_v.at[dst])

        plsc.subcore_barrier()

        def body(ci, carry):
            pltpu.async_copy(ew_v.at[ci], deg_sh.at[col_v.at[ci]], dsem,
                             add=True)
            return carry

        def drain(ci, carry):
            pltpu.make_async_copy(ew_v.at[ci], deg_sh.at[col_v.at[ci]],
                                  dsem).wait()
            return carry

        @pl.when(wid < NW - 1)
        def _():
            lax.fori_loop(0, slab, body, 0)
            lax.fori_loop(0, slab, drain, 0)

        @pl.when(wid == NW - 1)
        def _():
            lax.fori_loop(0, tail, body, 0)
            lax.fori_loop(0, tail, drain, 0)

        plsc.subcore_barrier()

        def replicate(off, cnt):
            pltpu.sync_copy(z2_hbm, rep)
            for i in range(q):
                pltpu.sync_copy(deg_sh.at[pl.ds(i * p + off, cnt)],
                                dloc.at[pl.ds(0, cnt)])

                def rep16(g, carry):
                    d16 = dloc[pl.ds(g * 16, 16)]
                    for j in range(16):
                        s = d16[j]
                        r = g * 16 + j
                        for k in range(h // 16):
                            sl = pl.ds(i * h + k * 16, 16)
                            rep[r, sl] = rep[r, sl] + s
                    return carry

                lax.fori_loop(0, -(-cnt // 16), rep16, 0)
            pltpu.sync_copy(rep.at[pl.ds(0, cnt)],
                            deg_out.at[cid, pl.ds(off, cnt)])

        @pl.when(sid < NS - 1)
        def _():
            replicate(sid * stile, stile)

        @pl.when(sid == NS - 1)
        def _():
            replicate((NS - 1) * stile, srem)

    return k_deg


def _make_k_prop(np_, h, slab, tail):
    q = 128 // h
    p = np_ // q
    stile = (-(-p // NS) + 7) // 8 * 8
    srem = p - stile * (NS - 1)
    assert 0 < srem <= stile

    @functools.partial(
        pl.kernel,
        out_type=jax.ShapeDtypeStruct((NC, p, 128), jnp.float32),
        mesh=_mesh(),
        scratch_types=[
            pltpu.VMEM((slab, CH), jnp.int32),
            pltpu.VMEM((slab, CH), jnp.int32),
            pltpu.VMEM((slab, CH), jnp.float32),
            [pltpu.VMEM((CH, h), jnp.float32)] * 4,
            pltpu.VMEM_SHARED((np_, h), jnp.float32),
            pltpu.VMEM_SHARED((np_, h), jnp.float32),
            [pltpu.SemaphoreType.DMA] * 4,
            [pltpu.SemaphoreType.DMA] * 4,
        ],
        compiler_params=pltpu.CompilerParams(use_tc_tiling_on_sc=False),
    )
    def k_prop(g_hbm, row_hbm, col_hbm, ew_hbm, zacc_hbm, out_hbm,
               row_v, col_v, ew_v, rows, acc_sh, g_sh, gsem, ssem):
        cid = lax.axis_index("c")
        sid = lax.axis_index("s")
        wid = sid * NC + cid

        def stage(off, cnt):
            pltpu.sync_copy(zacc_hbm.at[pl.ds(off * q, cnt * q)],
                            acc_sh.at[pl.ds(off * q, cnt * q)])
            for i in range(q):
                pltpu.sync_copy(g_hbm.at[pl.ds(off, cnt), pl.ds(i * h, h)],
                                g_sh.at[pl.ds(i * p + off, cnt)])

        @pl.when(sid < NS - 1)
        def _():
            stage(sid * stile, stile)

        @pl.when(sid == NS - 1)
        def _():
            stage((NS - 1) * stile, srem)

        @pl.when(wid < NW - 1)
        def _():
            sl = pl.ds(wid * slab, slab)
            pltpu.sync_copy(row_hbm.at[sl], row_v)
            pltpu.sync_copy(col_hbm.at[sl], col_v)
            pltpu.sync_copy(ew_hbm.at[sl], ew_v)

        @pl.when(wid == NW - 1)
        def _():
            sl = pl.ds((NW - 1) * slab, tail)
            dst = pl.ds(0, tail)
            pltpu.sync_copy(row_hbm.at[sl], row_v.at[dst])
            pltpu.sync_copy(col_hbm.at[sl], col_v.at[dst])
            pltpu.sync_copy(ew_hbm.at[sl], ew_v.at[dst])

        plsc.subcore_barrier()

        def scale(ci, rows_v):
            def scale16(q, c2):
                ew16 = ew_v[ci, pl.ds(q * 16, 16)]
                for j in range(16):
                    s = ew16[j]
                    r = q * 16 + j
                    for k in range(h // 16):
                        sl = pl.ds(k * 16, 16)
                        rows_v[r, sl] = rows_v[r, sl] * s
                return c2

            lax.fori_loop(0, CH // 16, scale16, 0)

        def gather(ci, b):
            pltpu.async_copy(g_sh.at[row_v.at[ci]], rows[b], gsem[b])

        def wait_gather(ci, b):
            pltpu.make_async_copy(g_sh.at[row_v.at[ci]], rows[b], gsem[b]).wait()

        def scatter(ci, b):
            pltpu.async_copy(rows[b], acc_sh.at[col_v.at[ci]], ssem[b],
                             add=True)

        def wait_scatter(ci, b):
            pltpu.make_async_copy(rows[b], acc_sh.at[col_v.at[ci]],
                                  ssem[b]).wait()

        def pipeline(nck):
            gather(0, 0)
            gather(1, 1)

            def body(i, carry):
                for k in range(4):
                    c = 4 * i + k
                    b = k
                    bn = (k + 2) % 4
                    wait_gather(c, b)
                    scale(c, rows[b])
                    scatter(c, b)

                    @pl.when(c + 2 < nck)
                    def _():
                        @pl.when(c >= 2)
                        def _():
                            wait_scatter(c - 2, bn)

                        gather(c + 2, bn)

                return carry

            lax.fori_loop(0, nck // 4, body, 0)
            for c in range(nck - 4, nck):
                wait_scatter(c, c % 4)

        @pl.when(wid < NW - 1)
        def _():
            pipeline(slab)

        @pl.when(wid == NW - 1)
        def _():
            pipeline(tail)

        plsc.subcore_barrier()

        def copy_out(off, cnt):
            for i in range(q):
                pltpu.sync_copy(acc_sh.at[pl.ds(i * p + off, cnt)],
                                out_hbm.at[cid, pl.ds(off, cnt),
                                           pl.ds(i * h, h)])

        @pl.when(sid < NS - 1)
        def _():
            copy_out(sid * stile, stile)

        @pl.when(sid == NS - 1)
        def _():
            copy_out((NS - 1) * stile, srem)

    return k_prop


def _tc1_body(n, np_, h, deg4_ref, x_ref, w1_ref, dis4_ref, g1_ref):
    q = 128 // h
    p = np_ // q
    dis4 = lax.rsqrt(deg4_ref[0] + deg4_ref[1] + 1.0)
    dis4_ref[...] = dis4
    x = x_ref[...]
    w1 = w1_ref[...]
    parts = []
    for i in range(q):
        lo = min(i * p, n)
        hi = min(i * p + p, n)
        hb = jnp.dot(x[lo:hi], w1, preferred_element_type=jnp.float32)
        parts.append(jnp.pad(hb, ((0, p - (hi - lo)), (0, 0))))
    h1p = jnp.concatenate(parts, axis=1)
    g1_ref[...] = dis4 * h1p


def _tc2_body(acc_ref, g1_ref, dis4_ref, b1_ref, w2_ref, g2_ref):
    dis4 = dis4_ref[...]
    pre = dis4 * (acc_ref[0] + acc_ref[1] + g1_ref[...]) + b1_ref[...]
    z = jnp.where(pre > 0, pre, NEG_SLOPE * pre)
    h2 = jnp.dot(z, w2_ref[...], preferred_element_type=jnp.float32)
    g2_ref[...] = h2 * dis4


def _tc3_body(acc_ref, g2_ref, dis4_ref, b2_ref, out_ref):
    out_ref[...] = (dis4_ref[...] * (acc_ref[0] + acc_ref[1] + g2_ref[...])
                    + b2_ref[...])


def kernel(x, edge_index, edge_weight, W1, b1, W2, b2):
    n, d = x.shape
    h = W1.shape[1]
    assert W2.shape[1] == h and 128 % h == 0
    e = edge_index.shape[1]
    q = 128 // h

    np_ = -(-n // (8 * q)) * (8 * q)
    p = np_ * h // 128
    assert n % q == 0

    chunks = -(-e // CH)
    slab = max(4, -(-(-(-chunks // NW)) // 4) * 4)
    tail = chunks - (NW - 1) * slab
    row_f = edge_index[0]
    col_f = edge_index[1]
    ew_f = edge_weight
    if e != chunks * CH or tail <= 0 or tail % 4 != 0:
        slab = max(4, -(-chunks // NW))
        slab = -(-slab // 4) * 4
        e_pad = NW * slab * CH
        tail = slab
        chunks = NW * slab
        row_f = jnp.pad(row_f, (0, e_pad - e))
        col_f = jnp.pad(col_f, (0, e_pad - e))
        ew_f = jnp.pad(ew_f, (0, e_pad - e))
    row_f = row_f.reshape(chunks, CH)
    col_f = col_f.reshape(chunks, CH)
    ew_f = ew_f.reshape(chunks, CH)

    zn = jnp.zeros((np_,), jnp.float32)
    zacc = jnp.zeros((np_, h), jnp.float32)
    stile = (-(-p // NS) + 7) // 8 * 8
    nbuf = -(-stile // 16) * 16
    z2 = jnp.zeros((nbuf, 128), jnp.float32)
    b1t = jnp.tile(b1, q).reshape(1, 128)
    b2t = jnp.tile(b2, q).reshape(1, 128)
    w2k = jnp.kron(jnp.eye(q, dtype=jnp.float32), W2)

    k_deg = _make_k_deg(np_, h, slab, tail)
    k_prop = _make_k_prop(np_, h, slab, tail)

    deg4 = k_deg(col_f, ew_f, zn, z2)

    k_tc1 = pl.pallas_call(
        functools.partial(_tc1_body, n, np_, h),
        out_shape=(jax.ShapeDtypeStruct((p, 128), jnp.float32),
                   jax.ShapeDtypeStruct((p, 128), jnp.float32)),
    )
    dis4, g1 = k_tc1(deg4, x, W1)

    acc1 = k_prop(g1, row_f, col_f, ew_f, zacc)

    k_tc2 = pl.pallas_call(
        _tc2_body,
        out_shape=jax.ShapeDtypeStruct((p, 128), jnp.float32),
    )
    g2 = k_tc2(acc1, g1, dis4, b1t, w2k)

    acc2 = k_prop(g2, row_f, col_f, ew_f, zacc)

    k_tc3 = pl.pallas_call(
        _tc3_body,
        out_shape=jax.ShapeDtypeStruct((p, 128), jnp.float32),
    )
    outp = k_tc3(acc2, g2, dis4, b2t)
    out = jnp.concatenate([outp[:, i * h:(i + 1) * h] for i in range(q)],
                          axis=0)
    return out[:n]
